# Initial kernel scaffold; baseline (speedup 1.0000x reference)
import jax
import jax.numpy as jnp
from jax.experimental import pallas as pl


def kernel(x, q_dw_w, q_dw_b, q_bn_g, q_bn_b, q_pw_w, q_pw_b, k_dw_w, k_dw_b, k_bn_g, k_bn_b, k_pw_w, k_pw_b, v_dw_w, v_dw_b, v_bn_g, v_bn_b, v_pw_w, v_pw_b, out_w, out_b):
    raise NotImplementedError("write your pallas kernel here")


# speedup vs baseline: 1.0159x; 1.0159x over previous
import math

import jax
import jax.numpy as jnp
from jax.experimental import pallas as pl
from jax.experimental.pallas import tpu as pltpu

_HEADS = 16
_QT = 8


def _body(x_ref, wq_ref, wk_ref, wv_ref, wo_ref, misc_ref, b_ref, out_ref,
          yq_ref, yk_ref, yv_ref):
    h = pl.program_id(0)
    dk = wq_ref.shape[0]
    T = x_ref.shape[0]
    qt = T // _QT

    @pl.when(h == 0)
    def _convs():
        D = x_ref.shape[1]
        zero = jnp.zeros((1, D), jnp.float32)
        for i in range(_QT):
            lo = i * qt
            xv = x_ref[lo:lo + qt, :]
            if i == 0:
                xm = jnp.concatenate([zero, x_ref[0:qt - 1, :]], axis=0)
            else:
                xm = x_ref[lo - 1:lo + qt - 1, :]
            if i == _QT - 1:
                xp = jnp.concatenate([x_ref[lo + 1:T, :], zero], axis=0)
            else:
                xp = x_ref[lo + 1:lo + qt + 1, :]
            for ref, base in ((yq_ref, 0), (yk_ref, 3), (yv_ref, 6)):
                w0 = misc_ref[base, :][None, :]
                w1 = misc_ref[base + 1, :][None, :]
                w2 = misc_ref[base + 2, :][None, :]
                ref[lo:lo + qt, :] = xm * w0 + xv * w1 + xp * w2

    bq = b_ref[0, 0, :][None, :]
    bk = b_ref[0, 1, :][None, :]
    bv = b_ref[0, 2, :][None, :]

    dims = (((1,), (1,)), ((), ()))
    k = jax.lax.dot_general(yk_ref[...], wk_ref[...], dims,
                            preferred_element_type=jnp.float32) + bk
    v = jax.lax.dot_general(yv_ref[...], wv_ref[...], dims,
                            preferred_element_type=jnp.float32) + bv

    def tile(i, carry):
        rows = pl.ds(i * qt, qt)
        q = jax.lax.dot_general(yq_ref[rows, :], wq_ref[...],
                                dims, preferred_element_type=jnp.float32) + bq
        scores = jax.lax.dot_general(q, k, dims,
                                     preferred_element_type=jnp.float32)
        m = jnp.max(scores, axis=1, keepdims=True)
        p = jnp.exp(scores - m)
        s = jnp.sum(p, axis=1, keepdims=True)
        ctx = jax.lax.dot_general(p, v, (((1,), (0,)), ((), ())),
                                  preferred_element_type=jnp.float32) / s
        contrib = jax.lax.dot_general(ctx, wo_ref[...],
                                      (((1,), (0,)), ((), ())),
                                      preferred_element_type=jnp.float32)

        @pl.when(h == 0)
        def _init():
            out_ref[rows, :] = contrib + misc_ref[9, :][None, :]

        @pl.when(h > 0)
        def _acc():
            out_ref[rows, :] += contrib

        return carry

    jax.lax.fori_loop(0, _QT, tile, 0)


def kernel(x, q_dw_w, q_dw_b, q_bn_g, q_bn_b, q_pw_w, q_pw_b,
           k_dw_w, k_dw_b, k_bn_g, k_bn_b, k_pw_w, k_pw_b,
           v_dw_w, v_dw_b, v_bn_g, v_bn_b, v_pw_w, v_pw_b,
           out_w, out_b):
    B, T, D = x.shape
    dk = D // _HEADS
    x2d = x[0]
    inv = 1.0 / math.sqrt(1.0 + 1e-5)

    def fold(pw_w, bn_g, bn_b, dw_b, pw_b):
        a = bn_g * inv
        w_eff = pw_w[:, :, 0] * a[None, :]
        b_eff = pw_w[:, :, 0] @ (dw_b * a + bn_b) + pw_b
        return w_eff, b_eff

    wq, bq = fold(q_pw_w, q_bn_g, q_bn_b, q_dw_b, q_pw_b)
    scale = 1.0 / math.sqrt(dk)
    wq = wq * scale
    bq = bq * scale
    wk, bk = fold(k_pw_w, k_bn_g, k_bn_b, k_dw_b, k_pw_b)
    wv, bv = fold(v_pw_w, v_bn_g, v_bn_b, v_dw_b, v_pw_b)

    rows = [q_dw_w[:, 0, 0], q_dw_w[:, 0, 1], q_dw_w[:, 0, 2],
            k_dw_w[:, 0, 0], k_dw_w[:, 0, 1], k_dw_w[:, 0, 2],
            v_dw_w[:, 0, 0], v_dw_w[:, 0, 1], v_dw_w[:, 0, 2],
            out_b]
    misc = jnp.stack(rows, axis=0)
    bias3 = jnp.stack([bq, bk, bv], 0).reshape(3, _HEADS, dk).transpose(1, 0, 2)

    out2d = pl.pallas_call(
        _body,
        grid=(_HEADS,),
        in_specs=[
            pl.BlockSpec((T, D), lambda h: (0, 0)),
            pl.BlockSpec((dk, D), lambda h: (h, 0)),
            pl.BlockSpec((dk, D), lambda h: (h, 0)),
            pl.BlockSpec((dk, D), lambda h: (h, 0)),
            pl.BlockSpec((dk, D), lambda h: (h, 0)),
            pl.BlockSpec((10, D), lambda h: (0, 0)),
            pl.BlockSpec((1, 3, dk), lambda h: (h, 0, 0)),
        ],
        out_specs=pl.BlockSpec((T, D), lambda h: (0, 0)),
        out_shape=jax.ShapeDtypeStruct((T, D), jnp.float32),
        scratch_shapes=[pltpu.VMEM((T, D), jnp.float32)] * 3,
        compiler_params=pltpu.CompilerParams(
            dimension_semantics=("arbitrary",)),
    )(x2d, wq, wk, wv, out_w.T, misc, bias3)

    return out2d[None, :, :]


# speedup vs baseline: 1.0450x; 1.0287x over previous
import math

import jax
import jax.numpy as jnp
from jax.experimental import pallas as pl
from jax.experimental.pallas import tpu as pltpu

_HEADS = 16
_QT = 8


def _body(x_ref, wq_ref, wk_ref, wv_ref, wo_ref, misc_ref, b_ref, out_ref,
          yq_ref, yk_ref, yv_ref):
    h = pl.program_id(0)
    dk = wq_ref.shape[0]
    T = x_ref.shape[0]
    qt = T // _QT

    @pl.when(h == 0)
    def _convs():
        D = x_ref.shape[1]
        zero = jnp.zeros((1, D), jnp.float32)
        for i in range(_QT):
            lo = i * qt
            xv = x_ref[lo:lo + qt, :]
            if i == 0:
                xm = jnp.concatenate([zero, x_ref[0:qt - 1, :]], axis=0)
            else:
                xm = x_ref[lo - 1:lo + qt - 1, :]
            if i == _QT - 1:
                xp = jnp.concatenate([x_ref[lo + 1:T, :], zero], axis=0)
            else:
                xp = x_ref[lo + 1:lo + qt + 1, :]
            for ref, base in ((yq_ref, 0), (yk_ref, 3), (yv_ref, 6)):
                w0 = misc_ref[base, :][None, :]
                w1 = misc_ref[base + 1, :][None, :]
                w2 = misc_ref[base + 2, :][None, :]
                ref[lo:lo + qt, :] = (
                    xm * w0 + xv * w1 + xp * w2).astype(jnp.bfloat16)

    bq = b_ref[0, 0, :][None, :]
    bk = b_ref[0, 1, :][None, :]
    bv = b_ref[0, 2, :][None, :]

    dims = (((1,), (1,)), ((), ()))
    k = (jax.lax.dot_general(yk_ref[...], wk_ref[...], dims,
                             preferred_element_type=jnp.float32)
         + bk).astype(jnp.bfloat16)
    v = (jax.lax.dot_general(yv_ref[...], wv_ref[...], dims,
                             preferred_element_type=jnp.float32)
         + bv).astype(jnp.bfloat16)

    def tile(i, carry):
        rows = pl.ds(i * qt, qt)
        q = (jax.lax.dot_general(yq_ref[rows, :], wq_ref[...], dims,
                                 preferred_element_type=jnp.float32)
             + bq).astype(jnp.bfloat16)
        scores = jax.lax.dot_general(q, k, dims,
                                     preferred_element_type=jnp.float32)
        m = jnp.max(scores, axis=1, keepdims=True)
        p = jnp.exp(scores - m).astype(jnp.bfloat16)
        s = jnp.sum(p.astype(jnp.float32), axis=1, keepdims=True)
        ctx = (jax.lax.dot_general(p, v, (((1,), (0,)), ((), ())),
                                   preferred_element_type=jnp.float32)
               / s).astype(jnp.bfloat16)
        contrib = jax.lax.dot_general(ctx, wo_ref[...],
                                      (((1,), (0,)), ((), ())),
                                      preferred_element_type=jnp.float32)

        @pl.when(h == 0)
        def _init():
            out_ref[rows, :] = contrib + misc_ref[9, :][None, :]

        @pl.when(h > 0)
        def _acc():
            out_ref[rows, :] += contrib

        return carry

    jax.lax.fori_loop(0, _QT, tile, 0)


def kernel(x, q_dw_w, q_dw_b, q_bn_g, q_bn_b, q_pw_w, q_pw_b,
           k_dw_w, k_dw_b, k_bn_g, k_bn_b, k_pw_w, k_pw_b,
           v_dw_w, v_dw_b, v_bn_g, v_bn_b, v_pw_w, v_pw_b,
           out_w, out_b):
    B, T, D = x.shape
    dk = D // _HEADS
    x2d = x[0]
    inv = 1.0 / math.sqrt(1.0 + 1e-5)

    def fold(pw_w, bn_g, bn_b, dw_b, pw_b):
        a = bn_g * inv
        w_eff = pw_w[:, :, 0] * a[None, :]
        b_eff = pw_w[:, :, 0] @ (dw_b * a + bn_b) + pw_b
        return w_eff, b_eff

    wq, bq = fold(q_pw_w, q_bn_g, q_bn_b, q_dw_b, q_pw_b)
    scale = 1.0 / math.sqrt(dk)
    wq = wq * scale
    bq = bq * scale
    wk, bk = fold(k_pw_w, k_bn_g, k_bn_b, k_dw_b, k_pw_b)
    wv, bv = fold(v_pw_w, v_bn_g, v_bn_b, v_dw_b, v_pw_b)

    rows = [q_dw_w[:, 0, 0], q_dw_w[:, 0, 1], q_dw_w[:, 0, 2],
            k_dw_w[:, 0, 0], k_dw_w[:, 0, 1], k_dw_w[:, 0, 2],
            v_dw_w[:, 0, 0], v_dw_w[:, 0, 1], v_dw_w[:, 0, 2],
            out_b]
    misc = jnp.stack(rows, axis=0)
    bias3 = jnp.stack([bq, bk, bv], 0).reshape(3, _HEADS, dk).transpose(1, 0, 2)

    out2d = pl.pallas_call(
        _body,
        grid=(_HEADS,),
        in_specs=[
            pl.BlockSpec((T, D), lambda h: (0, 0)),
            pl.BlockSpec((dk, D), lambda h: (h, 0)),
            pl.BlockSpec((dk, D), lambda h: (h, 0)),
            pl.BlockSpec((dk, D), lambda h: (h, 0)),
            pl.BlockSpec((dk, D), lambda h: (h, 0)),
            pl.BlockSpec((10, D), lambda h: (0, 0)),
            pl.BlockSpec((1, 3, dk), lambda h: (h, 0, 0)),
        ],
        out_specs=pl.BlockSpec((T, D), lambda h: (0, 0)),
        out_shape=jax.ShapeDtypeStruct((T, D), jnp.float32),
        scratch_shapes=[pltpu.VMEM((T, D), jnp.bfloat16)] * 3,
        compiler_params=pltpu.CompilerParams(
            dimension_semantics=("arbitrary",)),
    )(x2d, wq.astype(jnp.bfloat16), wk.astype(jnp.bfloat16),
      wv.astype(jnp.bfloat16), out_w.T.astype(jnp.bfloat16), misc, bias3)

    return out2d[None, :, :]
